# Initial kernel scaffold; baseline (speedup 1.0000x reference)
import jax
import jax.numpy as jnp
from jax.experimental import pallas as pl


def kernel(x, edge_index, Wl1, bl1, Wr1, Wl2, bl2, Wr2, Wl3, bl3, Wr3):
    raise NotImplementedError("write your pallas kernel here")


# speedup vs baseline: 1.2393x; 1.2393x over previous
import functools
import jax
import jax.numpy as jnp
from jax.experimental import pallas as pl
from jax.experimental.pallas import tpu as pltpu

N = 10000
E = 160000
N_PAD = 10240
BM = 512


def _fused_layer_body(seg_ref, cnt_ref, x_ref, wl_ref, wr_ref, b_ref, o_ref, *, relu):
    inv = 1.0 / jnp.maximum(cnt_ref[...], 1.0)
    mean = seg_ref[...] * inv
    acc = jnp.dot(mean, wl_ref[...], preferred_element_type=jnp.float32)
    acc = acc + jnp.dot(x_ref[...], wr_ref[...], preferred_element_type=jnp.float32)
    acc = acc + b_ref[...]
    if relu:
        acc = jnp.maximum(acc, 0.0)
    o_ref[...] = acc


def _tc_fused(seg, cnt, x, wl, wr, b, relu):
    k = x.shape[1]
    ko = wl.shape[1]
    grid = (N_PAD // BM,)
    return pl.pallas_call(
        functools.partial(_fused_layer_body, relu=relu),
        grid=grid,
        in_specs=[
            pl.BlockSpec((BM, k), lambda i: (i, 0)),
            pl.BlockSpec((BM, 1), lambda i: (i, 0)),
            pl.BlockSpec((BM, k), lambda i: (i, 0)),
            pl.BlockSpec((k, ko), lambda i: (0, 0)),
            pl.BlockSpec((k, ko), lambda i: (0, 0)),
            pl.BlockSpec((1, ko), lambda i: (0, 0)),
        ],
        out_specs=pl.BlockSpec((BM, ko), lambda i: (i, 0)),
        out_shape=jax.ShapeDtypeStruct((N_PAD, ko), jnp.float32),
    )(seg, cnt, x, wl, wr, b)


def _lin3_body(x_ref, wl_ref, wr_ref, b_ref, y_ref, r_ref):
    xv = x_ref[...]
    y_ref[...] = jnp.dot(xv, wl_ref[...], preferred_element_type=jnp.float32)
    r_ref[...] = jnp.dot(xv, wr_ref[...], preferred_element_type=jnp.float32) + b_ref[...]


def _tc_lin3(x, wl, wr, b):
    k = x.shape[1]
    ko = wl.shape[1]
    grid = (N_PAD // BM,)
    return pl.pallas_call(
        _lin3_body,
        grid=grid,
        in_specs=[
            pl.BlockSpec((BM, k), lambda i: (i, 0)),
            pl.BlockSpec((k, ko), lambda i: (0, 0)),
            pl.BlockSpec((k, ko), lambda i: (0, 0)),
            pl.BlockSpec((1, ko), lambda i: (0, 0)),
        ],
        out_specs=[
            pl.BlockSpec((BM, ko), lambda i: (i, 0)),
            pl.BlockSpec((BM, ko), lambda i: (i, 0)),
        ],
        out_shape=[
            jax.ShapeDtypeStruct((N_PAD, ko), jnp.float32),
            jax.ShapeDtypeStruct((N_PAD, ko), jnp.float32),
        ],
    )(x, wl, wr, b)


def _final_body(seg_ref, cnt_ref, r_ref, o_ref, *, d_valid):
    inv = 1.0 / jnp.maximum(cnt_ref[...], 1.0)
    z = seg_ref[...] * inv + r_ref[...]
    col = jax.lax.broadcasted_iota(jnp.int32, z.shape, 1)
    zm = jnp.where(col < d_valid, z, jnp.float32(-1e30))
    m = jnp.max(zm, axis=1, keepdims=True)
    s = jnp.sum(jnp.exp(zm - m), axis=1, keepdims=True)
    o_ref[...] = z - m - jnp.log(s)


def _tc_final(seg, cnt, r, d_valid):
    ko = seg.shape[1]
    grid = (N_PAD // BM,)
    return pl.pallas_call(
        functools.partial(_final_body, d_valid=d_valid),
        grid=grid,
        in_specs=[
            pl.BlockSpec((BM, ko), lambda i: (i, 0)),
            pl.BlockSpec((BM, 1), lambda i: (i, 0)),
            pl.BlockSpec((BM, ko), lambda i: (i, 0)),
        ],
        out_specs=pl.BlockSpec((BM, ko), lambda i: (i, 0)),
        out_shape=jax.ShapeDtypeStruct((N_PAD, ko), jnp.float32),
    )(seg, cnt, r)


def _segsum(feat, src, dst, n_out):
    return jax.ops.segment_sum(feat[src], dst, num_segments=n_out)


def kernel(x, edge_index, Wl1, bl1, Wr1, Wl2, bl2, Wr2, Wl3, bl3, Wr3):
    src = edge_index[0]
    dst = edge_index[1]

    cnt = jax.ops.segment_sum(jnp.ones((E,), jnp.float32), dst, num_segments=N)
    cnt_p = jnp.pad(cnt, (0, N_PAD - N)).reshape(N_PAD, 1)

    x_p = jnp.pad(x, ((0, N_PAD - N), (0, 0)))

    seg1 = jnp.pad(_segsum(x, src, dst, N), ((0, N_PAD - N), (0, 0)))
    h1 = _tc_fused(seg1, cnt_p, x_p, Wl1, Wr1, bl1.reshape(1, -1), relu=True)

    seg2 = jnp.pad(_segsum(h1[:N], src, dst, N), ((0, N_PAD - N), (0, 0)))
    h2 = _tc_fused(seg2, cnt_p, h1, Wl2, Wr2, bl2.reshape(1, -1), relu=True)

    d_out = Wl3.shape[1]
    d_pad = 64
    wl3p = jnp.pad(Wl3, ((0, 0), (0, d_pad - d_out)))
    wr3p = jnp.pad(Wr3, ((0, 0), (0, d_pad - d_out)))
    b3p = jnp.pad(bl3, (0, d_pad - d_out)).reshape(1, -1)
    y, r = _tc_lin3(h2, wl3p, wr3p, b3p)

    seg3 = jnp.pad(_segsum(y[:N], src, dst, N), ((0, N_PAD - N), (0, 0)))
    out = _tc_final(seg3, cnt_p, r, d_out)
    return out[:N, :d_out]


# speedup vs baseline: 1.2931x; 1.0435x over previous
import functools
import jax
import jax.numpy as jnp
from jax import lax
from jax.experimental import pallas as pl
from jax.experimental.pallas import tpu as pltpu
from jax.experimental.pallas import tpu_sc as plsc

N = 10000
E = 160000
N_PAD = 10240
BM = 512
RB = 256
EB = 128
E_PAD = 163840


def _sc_gather(feat, idx):
    n = idx.shape[0]
    cols = feat.shape[1]
    nb = n // (32 * EB)
    mesh = plsc.VectorSubcoreMesh(core_axis_name="c", subcore_axis_name="s")
    scratch = [
        pltpu.VMEM((EB,), jnp.int32),
        pltpu.VMEM((EB, cols), jnp.float32),
    ]

    def body(feat_hbm, idx_hbm, out_hbm, idx_v, rows_v):
        c = lax.axis_index("c")
        s = lax.axis_index("s")
        w = c * 16 + s
        for b in range(nb):
            off = (w * nb + b) * EB
            pltpu.sync_copy(idx_hbm.at[pl.ds(off, EB)], idx_v)
            pltpu.sync_copy(feat_hbm.at[idx_v], rows_v)
            pltpu.sync_copy(rows_v, out_hbm.at[pl.ds(off, EB)])

    f = pl.kernel(
        body,
        out_type=jax.ShapeDtypeStruct((n, cols), jnp.float32),
        mesh=mesh,
        scratch_types=scratch,
    )
    return f(feat, idx)


def _cumsum_body(tri_ref, m_ref, o_ref, carry_ref):
    g = pl.program_id(0)

    @pl.when(g == 0)
    def _():
        carry_ref[...] = jnp.zeros_like(carry_ref)

    blk = m_ref[...]
    excl = jnp.dot(tri_ref[...], blk, preferred_element_type=jnp.float32)
    o_ref[...] = carry_ref[...] + excl
    carry_ref[...] = carry_ref[...] + jnp.sum(blk, axis=0, keepdims=True)


def _tc_excl_cumsum(tri, m):
    cols = m.shape[1]
    grid = (E_PAD // RB,)
    return pl.pallas_call(
        _cumsum_body,
        grid=grid,
        in_specs=[
            pl.BlockSpec((RB, RB), lambda i: (0, 0)),
            pl.BlockSpec((RB, cols), lambda i: (i, 0)),
        ],
        out_specs=pl.BlockSpec((RB, cols), lambda i: (i, 0)),
        out_shape=jax.ShapeDtypeStruct((E_PAD, cols), jnp.float32),
        scratch_shapes=[pltpu.VMEM((1, cols), jnp.float32)],
    )(tri, m)


def _fused_layer_body(end_ref, start_ref, ie_ref, is_ref, x_ref, wl_ref, wr_ref,
                      b_ref, o_ref, *, relu):
    deg = (ie_ref[...] - is_ref[...]).astype(jnp.float32)
    inv = 1.0 / jnp.maximum(deg, 1.0)
    mean = (end_ref[...] - start_ref[...]) * inv
    acc = jnp.dot(mean, wl_ref[...], preferred_element_type=jnp.float32)
    acc = acc + jnp.dot(x_ref[...], wr_ref[...], preferred_element_type=jnp.float32)
    acc = acc + b_ref[...]
    if relu:
        acc = jnp.maximum(acc, 0.0)
    o_ref[...] = acc


def _tc_fused(end, start, ie, is_, x, wl, wr, b, relu):
    k = x.shape[1]
    ko = wl.shape[1]
    grid = (N_PAD // BM,)
    return pl.pallas_call(
        functools.partial(_fused_layer_body, relu=relu),
        grid=grid,
        in_specs=[
            pl.BlockSpec((BM, k), lambda i: (i, 0)),
            pl.BlockSpec((BM, k), lambda i: (i, 0)),
            pl.BlockSpec((BM, 1), lambda i: (i, 0)),
            pl.BlockSpec((BM, 1), lambda i: (i, 0)),
            pl.BlockSpec((BM, k), lambda i: (i, 0)),
            pl.BlockSpec((k, ko), lambda i: (0, 0)),
            pl.BlockSpec((k, ko), lambda i: (0, 0)),
            pl.BlockSpec((1, ko), lambda i: (0, 0)),
        ],
        out_specs=pl.BlockSpec((BM, ko), lambda i: (i, 0)),
        out_shape=jax.ShapeDtypeStruct((N_PAD, ko), jnp.float32),
    )(end, start, ie, is_, x, wl, wr, b)


def _lin3_body(x_ref, wl_ref, wr_ref, b_ref, y_ref, r_ref):
    xv = x_ref[...]
    y_ref[...] = jnp.dot(xv, wl_ref[...], preferred_element_type=jnp.float32)
    r_ref[...] = jnp.dot(xv, wr_ref[...], preferred_element_type=jnp.float32) + b_ref[...]


def _tc_lin3(x, wl, wr, b):
    k = x.shape[1]
    ko = wl.shape[1]
    grid = (N_PAD // BM,)
    return pl.pallas_call(
        _lin3_body,
        grid=grid,
        in_specs=[
            pl.BlockSpec((BM, k), lambda i: (i, 0)),
            pl.BlockSpec((k, ko), lambda i: (0, 0)),
            pl.BlockSpec((k, ko), lambda i: (0, 0)),
            pl.BlockSpec((1, ko), lambda i: (0, 0)),
        ],
        out_specs=[
            pl.BlockSpec((BM, ko), lambda i: (i, 0)),
            pl.BlockSpec((BM, ko), lambda i: (i, 0)),
        ],
        out_shape=[
            jax.ShapeDtypeStruct((N_PAD, ko), jnp.float32),
            jax.ShapeDtypeStruct((N_PAD, ko), jnp.float32),
        ],
    )(x, wl, wr, b)


def _final_body(end_ref, start_ref, ie_ref, is_ref, r_ref, o_ref, *, d_valid):
    deg = (ie_ref[...] - is_ref[...]).astype(jnp.float32)
    inv = 1.0 / jnp.maximum(deg, 1.0)
    z = (end_ref[...] - start_ref[...]) * inv + r_ref[...]
    col = jax.lax.broadcasted_iota(jnp.int32, z.shape, 1)
    zm = jnp.where(col < d_valid, z, jnp.float32(-1e30))
    m = jnp.max(zm, axis=1, keepdims=True)
    s = jnp.sum(jnp.exp(zm - m), axis=1, keepdims=True)
    o_ref[...] = z - m - jnp.log(s)


def _tc_final(end, start, ie, is_, r, d_valid):
    ko = end.shape[1]
    grid = (N_PAD // BM,)
    return pl.pallas_call(
        functools.partial(_final_body, d_valid=d_valid),
        grid=grid,
        in_specs=[
            pl.BlockSpec((BM, ko), lambda i: (i, 0)),
            pl.BlockSpec((BM, ko), lambda i: (i, 0)),
            pl.BlockSpec((BM, 1), lambda i: (i, 0)),
            pl.BlockSpec((BM, 1), lambda i: (i, 0)),
            pl.BlockSpec((BM, ko), lambda i: (i, 0)),
        ],
        out_specs=pl.BlockSpec((BM, ko), lambda i: (i, 0)),
        out_shape=jax.ShapeDtypeStruct((N_PAD, ko), jnp.float32),
    )(end, start, ie, is_, r)


def _segmean_parts(feat, src_s, bidx, tri):
    msgs = _sc_gather(feat, src_s)
    z = _tc_excl_cumsum(tri, msgs)
    bounds = _sc_gather(z, bidx)
    return bounds[:N_PAD], bounds[N_PAD:]


def kernel(x, edge_index, Wl1, bl1, Wr1, Wl2, bl2, Wr2, Wl3, bl3, Wr3):
    src = edge_index[0]
    dst = edge_index[1]

    perm = jnp.argsort(dst)
    dst_s = dst[perm]
    pad_e = E_PAD - E
    src_s = jnp.concatenate([src[perm], jnp.zeros((pad_e,), jnp.int32)])

    nodes = jnp.arange(N, dtype=jnp.int32)
    ie = jnp.searchsorted(dst_s, nodes, side="right").astype(jnp.int32)
    is_ = jnp.searchsorted(dst_s, nodes, side="left").astype(jnp.int32)
    ie_p = jnp.pad(ie, (0, N_PAD - N))
    is_p = jnp.pad(is_, (0, N_PAD - N))
    bidx = jnp.concatenate([ie_p, is_p])
    ie_c = ie_p.reshape(N_PAD, 1)
    is_c = is_p.reshape(N_PAD, 1)

    x_p = jnp.pad(x, ((0, N_PAD - N), (0, 0)))
    tri = jnp.tril(jnp.ones((RB, RB), jnp.float32), -1)

    end1, start1 = _segmean_parts(x, src_s, bidx, tri)
    h1 = _tc_fused(end1, start1, ie_c, is_c, x_p, Wl1, Wr1,
                   bl1.reshape(1, -1), relu=True)

    end2, start2 = _segmean_parts(h1, src_s, bidx, tri)
    h2 = _tc_fused(end2, start2, ie_c, is_c, h1, Wl2, Wr2,
                   bl2.reshape(1, -1), relu=True)

    d_out = Wl3.shape[1]
    d_pad = 128
    wl3p = jnp.pad(Wl3, ((0, 0), (0, d_pad - d_out)))
    wr3p = jnp.pad(Wr3, ((0, 0), (0, d_pad - d_out)))
    b3p = jnp.pad(bl3, (0, d_pad - d_out)).reshape(1, -1)
    y, r = _tc_lin3(h2, wl3p, wr3p, b3p)

    end3, start3 = _segmean_parts(y, src_s, bidx, tri)
    out = _tc_final(end3, start3, ie_c, is_c, r, d_out)
    return out[:N, :d_out]


# speedup vs baseline: 1.3751x; 1.0634x over previous
import functools
import jax
import jax.numpy as jnp
from jax import lax
from jax.experimental import pallas as pl
from jax.experimental.pallas import tpu as pltpu
from jax.experimental.pallas import tpu_sc as plsc

N = 10000
E = 160000
N_PAD = 10240
BM = 512
RB = 256
EB = 128
E_PAD = 163840


def _sc_gather(feat, idx):
    n = idx.shape[0]
    cols = feat.shape[1]
    eb = 128 if cols <= 256 else 64
    nb = n // (32 * eb)
    per_tile = nb * eb
    mesh = plsc.VectorSubcoreMesh(core_axis_name="c", subcore_axis_name="s")
    scratch = [
        pltpu.VMEM((per_tile,), jnp.int32),
        pltpu.VMEM((eb, cols), jnp.float32),
        pltpu.VMEM((eb, cols), jnp.float32),
        pltpu.SemaphoreType.DMA,
        pltpu.SemaphoreType.DMA,
        pltpu.SemaphoreType.DMA,
        pltpu.SemaphoreType.DMA,
    ]

    def body(feat_hbm, idx_hbm, out_hbm, idx_v, r0, r1, gs0, gs1, ws0, ws1):
        c = lax.axis_index("c")
        s = lax.axis_index("s")
        w = c * 16 + s
        base = w * per_tile
        pltpu.sync_copy(idx_hbm.at[pl.ds(base, per_tile)], idx_v)
        bufs = (r0, r1)
        gsems = (gs0, gs1)
        wsems = (ws0, ws1)

        def g_start(b):
            return pltpu.async_copy(
                feat_hbm.at[idx_v.at[pl.ds(b * eb, eb)]], bufs[b % 2], gsems[b % 2])

        gd = [None] * nb
        wd = [None] * nb
        gd[0] = g_start(0)
        for b in range(nb):
            if b + 1 < nb:
                if b - 1 >= 0:
                    wd[b - 1].wait()
                gd[b + 1] = g_start(b + 1)
            gd[b].wait()
            wd[b] = pltpu.async_copy(
                bufs[b % 2], out_hbm.at[pl.ds(base + b * eb, eb)], wsems[b % 2])
        if nb >= 2:
            wd[nb - 2].wait()
        wd[nb - 1].wait()

    f = pl.kernel(
        body,
        out_type=jax.ShapeDtypeStruct((n, cols), jnp.float32),
        mesh=mesh,
        scratch_types=scratch,
    )
    return f(feat, idx)


def _cumsum_body(tri_ref, m_ref, o_ref, carry_ref):
    g = pl.program_id(0)

    @pl.when(g == 0)
    def _():
        carry_ref[...] = jnp.zeros_like(carry_ref)

    blk = m_ref[...]
    excl = jnp.dot(tri_ref[...], blk, preferred_element_type=jnp.float32)
    o_ref[...] = carry_ref[...] + excl
    carry_ref[...] = carry_ref[...] + jnp.sum(blk, axis=0, keepdims=True)


def _tc_excl_cumsum(tri, m):
    cols = m.shape[1]
    grid = (E_PAD // RB,)
    return pl.pallas_call(
        _cumsum_body,
        grid=grid,
        in_specs=[
            pl.BlockSpec((RB, RB), lambda i: (0, 0)),
            pl.BlockSpec((RB, cols), lambda i: (i, 0)),
        ],
        out_specs=pl.BlockSpec((RB, cols), lambda i: (i, 0)),
        out_shape=jax.ShapeDtypeStruct((E_PAD, cols), jnp.float32),
        scratch_shapes=[pltpu.VMEM((1, cols), jnp.float32)],
    )(tri, m)


def _fused_layer_body(end_ref, start_ref, ie_ref, is_ref, x_ref, wl_ref, wr_ref,
                      b_ref, o_ref, *, relu):
    deg = (ie_ref[...] - is_ref[...]).astype(jnp.float32)
    inv = 1.0 / jnp.maximum(deg, 1.0)
    mean = (end_ref[...] - start_ref[...]) * inv
    acc = jnp.dot(mean, wl_ref[...], preferred_element_type=jnp.float32)
    acc = acc + jnp.dot(x_ref[...], wr_ref[...], preferred_element_type=jnp.float32)
    acc = acc + b_ref[...]
    if relu:
        acc = jnp.maximum(acc, 0.0)
    o_ref[...] = acc


def _tc_fused(end, start, ie, is_, x, wl, wr, b, relu):
    k = x.shape[1]
    ko = wl.shape[1]
    grid = (N_PAD // BM,)
    return pl.pallas_call(
        functools.partial(_fused_layer_body, relu=relu),
        grid=grid,
        in_specs=[
            pl.BlockSpec((BM, k), lambda i: (i, 0)),
            pl.BlockSpec((BM, k), lambda i: (i, 0)),
            pl.BlockSpec((BM, 1), lambda i: (i, 0)),
            pl.BlockSpec((BM, 1), lambda i: (i, 0)),
            pl.BlockSpec((BM, k), lambda i: (i, 0)),
            pl.BlockSpec((k, ko), lambda i: (0, 0)),
            pl.BlockSpec((k, ko), lambda i: (0, 0)),
            pl.BlockSpec((1, ko), lambda i: (0, 0)),
        ],
        out_specs=pl.BlockSpec((BM, ko), lambda i: (i, 0)),
        out_shape=jax.ShapeDtypeStruct((N_PAD, ko), jnp.float32),
    )(end, start, ie, is_, x, wl, wr, b)


def _lin3_body(x_ref, wl_ref, wr_ref, b_ref, y_ref, r_ref):
    xv = x_ref[...]
    y_ref[...] = jnp.dot(xv, wl_ref[...], preferred_element_type=jnp.float32)
    r_ref[...] = jnp.dot(xv, wr_ref[...], preferred_element_type=jnp.float32) + b_ref[...]


def _tc_lin3(x, wl, wr, b):
    k = x.shape[1]
    ko = wl.shape[1]
    grid = (N_PAD // BM,)
    return pl.pallas_call(
        _lin3_body,
        grid=grid,
        in_specs=[
            pl.BlockSpec((BM, k), lambda i: (i, 0)),
            pl.BlockSpec((k, ko), lambda i: (0, 0)),
            pl.BlockSpec((k, ko), lambda i: (0, 0)),
            pl.BlockSpec((1, ko), lambda i: (0, 0)),
        ],
        out_specs=[
            pl.BlockSpec((BM, ko), lambda i: (i, 0)),
            pl.BlockSpec((BM, ko), lambda i: (i, 0)),
        ],
        out_shape=[
            jax.ShapeDtypeStruct((N_PAD, ko), jnp.float32),
            jax.ShapeDtypeStruct((N_PAD, ko), jnp.float32),
        ],
    )(x, wl, wr, b)


def _final_body(end_ref, start_ref, ie_ref, is_ref, r_ref, o_ref, *, d_valid):
    deg = (ie_ref[...] - is_ref[...]).astype(jnp.float32)
    inv = 1.0 / jnp.maximum(deg, 1.0)
    z = (end_ref[...] - start_ref[...]) * inv + r_ref[...]
    col = jax.lax.broadcasted_iota(jnp.int32, z.shape, 1)
    zm = jnp.where(col < d_valid, z, jnp.float32(-1e30))
    m = jnp.max(zm, axis=1, keepdims=True)
    s = jnp.sum(jnp.exp(zm - m), axis=1, keepdims=True)
    o_ref[...] = z - m - jnp.log(s)


def _tc_final(end, start, ie, is_, r, d_valid):
    ko = end.shape[1]
    grid = (N_PAD // BM,)
    return pl.pallas_call(
        functools.partial(_final_body, d_valid=d_valid),
        grid=grid,
        in_specs=[
            pl.BlockSpec((BM, ko), lambda i: (i, 0)),
            pl.BlockSpec((BM, ko), lambda i: (i, 0)),
            pl.BlockSpec((BM, 1), lambda i: (i, 0)),
            pl.BlockSpec((BM, 1), lambda i: (i, 0)),
            pl.BlockSpec((BM, ko), lambda i: (i, 0)),
        ],
        out_specs=pl.BlockSpec((BM, ko), lambda i: (i, 0)),
        out_shape=jax.ShapeDtypeStruct((N_PAD, ko), jnp.float32),
    )(end, start, ie, is_, r)


def _segmean_parts(feat, src_s, bidx, tri):
    msgs = _sc_gather(feat, src_s)
    z = _tc_excl_cumsum(tri, msgs)
    bounds = _sc_gather(z, bidx)
    return bounds[:N_PAD], bounds[N_PAD:]


def kernel(x, edge_index, Wl1, bl1, Wr1, Wl2, bl2, Wr2, Wl3, bl3, Wr3):
    src = edge_index[0]
    dst = edge_index[1]

    perm = jnp.argsort(dst)
    dst_s = dst[perm]
    pad_e = E_PAD - E
    src_s = jnp.concatenate([src[perm], jnp.zeros((pad_e,), jnp.int32)])

    nodes = jnp.arange(N, dtype=jnp.int32)
    ie = jnp.searchsorted(dst_s, nodes, side="right").astype(jnp.int32)
    is_ = jnp.searchsorted(dst_s, nodes, side="left").astype(jnp.int32)
    ie_p = jnp.pad(ie, (0, N_PAD - N))
    is_p = jnp.pad(is_, (0, N_PAD - N))
    bidx = jnp.concatenate([ie_p, is_p])
    ie_c = ie_p.reshape(N_PAD, 1)
    is_c = is_p.reshape(N_PAD, 1)

    x_p = jnp.pad(x, ((0, N_PAD - N), (0, 0)))
    tri = jnp.tril(jnp.ones((RB, RB), jnp.float32), -1)

    end1, start1 = _segmean_parts(x, src_s, bidx, tri)
    h1 = _tc_fused(end1, start1, ie_c, is_c, x_p, Wl1, Wr1,
                   bl1.reshape(1, -1), relu=True)

    end2, start2 = _segmean_parts(h1, src_s, bidx, tri)
    h2 = _tc_fused(end2, start2, ie_c, is_c, h1, Wl2, Wr2,
                   bl2.reshape(1, -1), relu=True)

    d_out = Wl3.shape[1]
    d_pad = 128
    wl3p = jnp.pad(Wl3, ((0, 0), (0, d_pad - d_out)))
    wr3p = jnp.pad(Wr3, ((0, 0), (0, d_pad - d_out)))
    b3p = jnp.pad(bl3, (0, d_pad - d_out)).reshape(1, -1)
    y, r = _tc_lin3(h2, wl3p, wr3p, b3p)

    end3, start3 = _segmean_parts(y, src_s, bidx, tri)
    out = _tc_final(end3, start3, ie_c, is_c, r, d_out)
    return out[:N, :d_out]
